# baseline (speedup 1.0000x reference)
import jax
import jax.numpy as jnp
from jax.experimental import pallas as pl
from jax.experimental.pallas import tpu as pltpu

_B, _T, _S, _D = 8, 16, 576, 768


def _body(pref, x_hbm, tok_ref, o_hbm, tok_frame, sem):
    tok_frame[...] = jnp.broadcast_to(tok_ref[...], (_S, _D))
    for b in range(_B):
        p = pref[b]
        for t in range(_T):
            masked = t < p

            @pl.when(masked)
            def _():
                pltpu.make_async_copy(tok_frame, o_hbm.at[b, t], sem).start()

            @pl.when(jnp.logical_not(masked))
            def _():
                pltpu.make_async_copy(x_hbm.at[b, t], o_hbm.at[b, t], sem).start()

    for b in range(_B):
        for t in range(_T):
            pltpu.make_async_copy(x_hbm.at[b, t], o_hbm.at[b, t], sem).wait()


def kernel(x, prefix_len, dropout_mask_token):
    tok2d = dropout_mask_token.reshape(1, _D)
    grid_spec = pltpu.PrefetchScalarGridSpec(
        num_scalar_prefetch=1,
        grid=(1,),
        in_specs=[
            pl.BlockSpec(memory_space=pl.ANY),
            pl.BlockSpec((1, _D), lambda i, pref: (0, 0)),
        ],
        out_specs=pl.BlockSpec(memory_space=pl.ANY),
        scratch_shapes=[
            pltpu.VMEM((_S, _D), jnp.float32),
            pltpu.SemaphoreType.DMA,
        ],
    )
    fn = pl.pallas_call(
        _body,
        grid_spec=grid_spec,
        out_shape=jax.ShapeDtypeStruct(x.shape, x.dtype),
    )
    return fn(prefix_len, x, tok2d)


# speedup vs baseline: 24.2198x; 24.2198x over previous
import jax
import jax.numpy as jnp
from jax.experimental import pallas as pl
from jax.experimental.pallas import tpu as pltpu

_B, _T, _S, _D = 8, 16, 576, 768
_TB = 4


def _body(pref, x_ref, tok_ref, o_ref):
    b = pl.program_id(0)
    j = pl.program_id(1)
    p = pref[b]
    tids = j * _TB + jax.lax.broadcasted_iota(jnp.int32, (1, _TB, 1, 1), 1)
    mask = tids < p
    o_ref[...] = jnp.where(mask, tok_ref[...][None, None, :, :], x_ref[...])


def _x_index_map(b, j, pref):
    p = pref[b]
    fully_masked = j * _TB + _TB - 1 < p
    j_in = jnp.where(fully_masked, jnp.minimum(p // _TB, _T // _TB - 1), j)
    return b, j_in, 0, 0


def kernel(x, prefix_len, dropout_mask_token):
    tok2d = dropout_mask_token.reshape(1, _D)
    grid_spec = pltpu.PrefetchScalarGridSpec(
        num_scalar_prefetch=1,
        grid=(_B, _T // _TB),
        in_specs=[
            pl.BlockSpec((1, _TB, _S, _D), _x_index_map),
            pl.BlockSpec((1, _D), lambda b, j, pref: (0, 0)),
        ],
        out_specs=pl.BlockSpec((1, _TB, _S, _D), lambda b, j, pref: (b, j, 0, 0)),
    )
    fn = pl.pallas_call(
        _body,
        grid_spec=grid_spec,
        out_shape=jax.ShapeDtypeStruct(x.shape, x.dtype),
    )
    return fn(prefix_len, x, tok2d)


# speedup vs baseline: 24.3581x; 1.0057x over previous
import jax
import jax.numpy as jnp
from jax.experimental import pallas as pl
from jax.experimental.pallas import tpu as pltpu

_B, _T, _S, _D = 8, 16, 576, 768
_TB = 4


def _body(pref, x_ref, tok_ref, o_ref):
    b = pl.program_id(0)
    j = pl.program_id(1)
    p = pref[b]
    for tt in range(_TB):
        masked = j * _TB + tt < p

        @pl.when(masked)
        def _():
            o_ref[0, tt] = jnp.broadcast_to(tok_ref[...], (_S, _D))

        @pl.when(jnp.logical_not(masked))
        def _():
            o_ref[0, tt] = x_ref[0, tt]


def _x_index_map(b, j, pref):
    p = pref[b]
    fully_masked = j * _TB + _TB - 1 < p
    j_in = jnp.where(fully_masked, jnp.minimum(p // _TB, _T // _TB - 1), j)
    return b, j_in, 0, 0


def kernel(x, prefix_len, dropout_mask_token):
    tok2d = dropout_mask_token.reshape(1, _D)
    grid_spec = pltpu.PrefetchScalarGridSpec(
        num_scalar_prefetch=1,
        grid=(_B, _T // _TB),
        in_specs=[
            pl.BlockSpec((1, _TB, _S, _D), _x_index_map),
            pl.BlockSpec((1, _D), lambda b, j, pref: (0, 0)),
        ],
        out_specs=pl.BlockSpec((1, _TB, _S, _D), lambda b, j, pref: (b, j, 0, 0)),
    )
    fn = pl.pallas_call(
        _body,
        grid_spec=grid_spec,
        out_shape=jax.ShapeDtypeStruct(x.shape, x.dtype),
    )
    return fn(prefix_len, x, tok2d)
